# baseline (speedup 1.0000x reference)
import functools
import jax
import jax.numpy as jnp
from jax import lax
from jax.experimental import pallas as pl
from jax.experimental.pallas import tpu as pltpu
from jax.experimental.pallas import tpu_sc as plsc

VOCAB = 1000000
EMBED = 64
WIDE = 128
MAX_LEN = 200
BATCH = 4096

NC, NS, LANES = 2, 16, 16
NW = NC * NS
ROWS = BATCH * MAX_LEN
RPW = ROWS // NW
CHUNK = 128
NCHUNK = RPW // CHUNK
NBUF = 4
NOB = 2
NGRP = NCHUNK // NBUF
SCALE = 8.0
NSL = EMBED // LANES

WPW = 31248
WCH = 168
WNCH = WPW // WCH
WREM = VOCAB - NW * WPW

_mesh = plsc.VectorSubcoreMesh(core_axis_name="c", subcore_axis_name="s")


@functools.partial(
    pl.kernel,
    out_type=jax.ShapeDtypeStruct((VOCAB, WIDE), jnp.float32),
    mesh=_mesh,
    scratch_types=[
        [pltpu.VMEM((WCH, EMBED), jnp.float32) for _ in range(2)],
        [pltpu.VMEM((WCH, WIDE), jnp.float32) for _ in range(2)],
        [pltpu.SemaphoreType.DMA for _ in range(2)],
        [pltpu.SemaphoreType.DMA for _ in range(2)],
    ],
)
def _widen_kernel(table_hbm, wide_hbm, src_v, dst_v, sem_r, sem_w):
    wid = lax.axis_index("s") * NC + lax.axis_index("c")
    base0 = wid * WPW

    def read(chunk, buf, n=WCH):
        pltpu.async_copy(table_hbm.at[pl.ds(base0 + chunk * WCH, n)],
                         src_v[buf].at[pl.ds(0, n)], sem_r[buf])

    def wait_read(buf, n=WCH):
        pltpu.make_async_copy(table_hbm.at[pl.ds(0, n)],
                              src_v[buf].at[pl.ds(0, n)], sem_r[buf]).wait()

    def repack(buf, n=WCH):
        def row(r, carry):
            for d in range(NSL):
                sl = pl.ds(d * LANES, LANES)
                dst_v[buf][r, sl] = src_v[buf][r, sl]
            return carry
        lax.fori_loop(0, n, row, 0, unroll=8)

    def write(chunk, buf, n=WCH):
        pltpu.async_copy(dst_v[buf].at[pl.ds(0, n)],
                         wide_hbm.at[pl.ds(base0 + chunk * WCH, n)],
                         sem_w[buf])

    def wait_write(buf, n=WCH):
        pltpu.make_async_copy(dst_v[buf].at[pl.ds(0, n)],
                              wide_hbm.at[pl.ds(0, n)], sem_w[buf]).wait()

    read(0, 0)

    def body(c, carry):
        for b in range(2):
            cc = c * 2 + b
            nxt = 1 - b

            @pl.when(cc + 1 < WNCH)
            def _():
                read(cc + 1, nxt)
            wait_read(b)

            @pl.when(cc >= 2)
            def _():
                wait_write(b)
            repack(b)
            write(cc, b)
        return carry

    lax.fori_loop(0, WNCH // 2, body, 0, unroll=False)
    wait_write(0)
    wait_write(1)

    @pl.when(wid == NW - 1)
    def _():
        base = NW * WPW - base0
        pltpu.async_copy(table_hbm.at[pl.ds(base0 + base, WREM)],
                         src_v[0].at[pl.ds(0, WREM)], sem_r[0])
        pltpu.make_async_copy(table_hbm.at[pl.ds(0, WREM)],
                              src_v[0].at[pl.ds(0, WREM)], sem_r[0]).wait()

        def row(r, carry):
            for d in range(NSL):
                sl = pl.ds(d * LANES, LANES)
                dst_v[0][r, sl] = src_v[0][r, sl]
            return carry
        lax.fori_loop(0, WREM, row, 0, unroll=8)
        pltpu.async_copy(dst_v[0].at[pl.ds(0, WREM)],
                         wide_hbm.at[pl.ds(base0 + base, WREM)], sem_w[0])
        pltpu.make_async_copy(dst_v[0].at[pl.ds(0, WREM)],
                              wide_hbm.at[pl.ds(0, WREM)], sem_w[0]).wait()


@functools.partial(
    pl.kernel,
    out_type=jax.ShapeDtypeStruct((ROWS, EMBED), jnp.float32),
    mesh=_mesh,
    scratch_types=[
        [pltpu.VMEM((CHUNK,), jnp.int32) for _ in range(NBUF)],
        [pltpu.VMEM((CHUNK, WIDE), jnp.float32) for _ in range(NBUF)],
        [pltpu.VMEM((CHUNK, EMBED), jnp.float32) for _ in range(NOB)],
        pltpu.VMEM(((MAX_LEN + CHUNK) * EMBED,), jnp.float32),
        [pltpu.SemaphoreType.DMA for _ in range(NBUF)],
        [pltpu.SemaphoreType.DMA for _ in range(NBUF)],
        [pltpu.SemaphoreType.DMA for _ in range(NOB)],
    ],
)
def _embed_kernel(idx_hbm, pos_hbm, wide_hbm, out_hbm,
                  idx_v, rows_v, out_v, pos_v, sem_i, sem_g, sem_w):
    wid = lax.axis_index("s") * NC + lax.axis_index("c")
    base0 = wid * RPW
    pltpu.sync_copy(pos_hbm, pos_v)

    def fetch_idx(chunk, buf):
        pltpu.async_copy(idx_hbm.at[pl.ds(base0 + chunk * CHUNK, CHUNK)],
                         idx_v[buf], sem_i[buf])

    def wait_idx(buf):
        pltpu.make_async_copy(idx_hbm.at[pl.ds(0, CHUNK)], idx_v[buf],
                              sem_i[buf]).wait()

    def gather(buf):
        pltpu.async_copy(wide_hbm.at[idx_v[buf]], rows_v[buf], sem_g[buf])

    def wait_gather(buf):
        pltpu.make_async_copy(wide_hbm.at[pl.ds(0, CHUNK)], rows_v[buf],
                              sem_g[buf]).wait()

    def write_out(chunk, ob):
        pltpu.async_copy(out_v[ob],
                         out_hbm.at[pl.ds(base0 + chunk * CHUNK, CHUNK)],
                         sem_w[ob])

    def wait_write(ob):
        pltpu.make_async_copy(out_v[ob], out_hbm.at[pl.ds(0, CHUNK)],
                              sem_w[ob]).wait()

    fetch_idx(0, 0)
    wait_idx(0)
    gather(0)
    fetch_idx(1, 1)

    def group_body(p, carry):
        for b in range(NBUF):
            c = p * NBUF + b
            nxt = (b + 1) % NBUF
            nxt2 = (b + 2) % NBUF
            ob = b % NOB

            @pl.when(c + 1 < NCHUNK)
            def _():
                wait_idx(nxt)
                gather(nxt)

            @pl.when(c + 2 < NCHUNK)
            def _():
                fetch_idx(c + 2, nxt2)

            wait_gather(b)

            @pl.when(c >= NOB)
            def _():
                wait_write(ob)

            p0 = lax.rem(c * CHUNK, MAX_LEN)

            def row_body(r, rcarry):
                poff = (p0 + r) * EMBED
                for d in range(NSL):
                    sl = pl.ds(d * LANES, LANES)
                    out_v[ob][r, sl] = (
                        rows_v[b][r, sl] * SCALE
                        + pos_v[pl.ds(poff + d * LANES, LANES)])
                return rcarry

            lax.fori_loop(0, CHUNK, row_body, 0, unroll=8)

            write_out(c, ob)
        return carry

    lax.fori_loop(0, NGRP, group_body, 0, unroll=False)

    for ob in range(NOB):
        wait_write(ob)


def kernel(to_emb, token_table, pos_table):
    idx = to_emb.reshape(ROWS)
    pos_rep = (jnp.concatenate([pos_table] * 3, axis=0)[:MAX_LEN + CHUNK]
               .reshape(-1))
    wide = _widen_kernel(token_table)
    out = _embed_kernel(idx, pos_rep, wide)
    return out.reshape(BATCH, MAX_LEN, EMBED)


# speedup vs baseline: 1.0008x; 1.0008x over previous
import functools
import jax
import jax.numpy as jnp
from jax import lax
from jax.experimental import pallas as pl
from jax.experimental.pallas import tpu as pltpu
from jax.experimental.pallas import tpu_sc as plsc

VOCAB = 1000000
EMBED = 64
WIDE = 128
MAX_LEN = 200
BATCH = 4096

NC, NS, LANES = 2, 16, 16
NW = NC * NS
ROWS = BATCH * MAX_LEN
RPW = ROWS // NW
CHUNK = 128
NCHUNK = RPW // CHUNK
NBUF = 4
NOB = 2
NGRP = NCHUNK // NBUF
SCALE = 8.0
NSL = EMBED // LANES

WPW = 31248
WCH = 168
WNCH = WPW // WCH
WREM = VOCAB - NW * WPW

_mesh = plsc.VectorSubcoreMesh(core_axis_name="c", subcore_axis_name="s")


@functools.partial(
    pl.kernel,
    out_type=jax.ShapeDtypeStruct((VOCAB, WIDE), jnp.float32),
    mesh=_mesh,
    scratch_types=[
        [pltpu.VMEM((WCH, EMBED), jnp.float32) for _ in range(2)],
        [pltpu.VMEM((WCH, WIDE), jnp.float32) for _ in range(2)],
        [pltpu.SemaphoreType.DMA for _ in range(2)],
        [pltpu.SemaphoreType.DMA for _ in range(2)],
    ],
)
def _widen_kernel(table_hbm, wide_hbm, src_v, dst_v, sem_r, sem_w):
    wid = lax.axis_index("s") * NC + lax.axis_index("c")
    base0 = wid * WPW

    def read(chunk, buf, n=WCH):
        pltpu.async_copy(table_hbm.at[pl.ds(base0 + chunk * WCH, n)],
                         src_v[buf].at[pl.ds(0, n)], sem_r[buf])

    def wait_read(buf, n=WCH):
        pltpu.make_async_copy(table_hbm.at[pl.ds(0, n)],
                              src_v[buf].at[pl.ds(0, n)], sem_r[buf]).wait()

    def repack(buf, n=WCH):
        def row(r, carry):
            for d in range(NSL):
                sl = pl.ds(d * LANES, LANES)
                dst_v[buf][r, sl] = src_v[buf][r, sl]
            return carry
        lax.fori_loop(0, n, row, 0, unroll=8)

    def write(chunk, buf, n=WCH):
        pltpu.async_copy(dst_v[buf].at[pl.ds(0, n)],
                         wide_hbm.at[pl.ds(base0 + chunk * WCH, n)],
                         sem_w[buf])

    def wait_write(buf, n=WCH):
        pltpu.make_async_copy(dst_v[buf].at[pl.ds(0, n)],
                              wide_hbm.at[pl.ds(0, n)], sem_w[buf]).wait()

    read(0, 0)

    def body(c, carry):
        for b in range(2):
            cc = c * 2 + b
            nxt = 1 - b

            @pl.when(cc + 1 < WNCH)
            def _():
                read(cc + 1, nxt)
            wait_read(b)

            @pl.when(cc >= 2)
            def _():
                wait_write(b)
            repack(b)
            write(cc, b)
        return carry

    lax.fori_loop(0, WNCH // 2, body, 0, unroll=False)
    wait_write(0)
    wait_write(1)

    @pl.when(wid == NW - 1)
    def _():
        pltpu.async_copy(table_hbm.at[pl.ds(base0 + WPW, WREM)],
                         src_v[0].at[pl.ds(0, WREM)], sem_r[0])
        pltpu.make_async_copy(table_hbm.at[pl.ds(0, WREM)],
                              src_v[0].at[pl.ds(0, WREM)], sem_r[0]).wait()

        def row(r, carry):
            for d in range(NSL):
                sl = pl.ds(d * LANES, LANES)
                dst_v[0][r, sl] = src_v[0][r, sl]
            return carry
        lax.fori_loop(0, WREM, row, 0, unroll=8)
        pltpu.async_copy(dst_v[0].at[pl.ds(0, WREM)],
                         wide_hbm.at[pl.ds(base0 + WPW, WREM)], sem_w[0])
        pltpu.make_async_copy(dst_v[0].at[pl.ds(0, WREM)],
                              wide_hbm.at[pl.ds(0, WREM)], sem_w[0]).wait()


@functools.partial(
    pl.kernel,
    out_type=jax.ShapeDtypeStruct((ROWS, EMBED), jnp.float32),
    mesh=_mesh,
    scratch_types=[
        [pltpu.VMEM((CHUNK,), jnp.int32) for _ in range(NBUF)],
        [pltpu.VMEM((CHUNK, WIDE), jnp.float32) for _ in range(NBUF)],
        [pltpu.VMEM((CHUNK, EMBED), jnp.float32) for _ in range(NOB)],
        pltpu.VMEM(((MAX_LEN + CHUNK) * EMBED,), jnp.float32),
        [pltpu.SemaphoreType.DMA for _ in range(NBUF)],
        [pltpu.SemaphoreType.DMA for _ in range(NBUF)],
        [pltpu.SemaphoreType.DMA for _ in range(NOB)],
    ],
)
def _embed_kernel(idx_hbm, pos_hbm, wide_hbm, out_hbm,
                  idx_v, rows_v, out_v, pos_v, sem_i, sem_g, sem_w):
    wid = lax.axis_index("s") * NC + lax.axis_index("c")
    base0 = wid * RPW
    pltpu.sync_copy(pos_hbm, pos_v)

    def fetch_idx(chunk, buf):
        pltpu.async_copy(idx_hbm.at[pl.ds(base0 + chunk * CHUNK, CHUNK)],
                         idx_v[buf], sem_i[buf])

    def wait_idx(buf):
        pltpu.make_async_copy(idx_hbm.at[pl.ds(0, CHUNK)], idx_v[buf],
                              sem_i[buf]).wait()

    def gather(buf):
        pltpu.async_copy(wide_hbm.at[idx_v[buf]], rows_v[buf], sem_g[buf])

    def wait_gather(buf):
        pltpu.make_async_copy(wide_hbm.at[pl.ds(0, CHUNK)], rows_v[buf],
                              sem_g[buf]).wait()

    def write_out(chunk, ob):
        pltpu.async_copy(out_v[ob],
                         out_hbm.at[pl.ds(base0 + chunk * CHUNK, CHUNK)],
                         sem_w[ob])

    def wait_write(ob):
        pltpu.make_async_copy(out_v[ob], out_hbm.at[pl.ds(0, CHUNK)],
                              sem_w[ob]).wait()

    fetch_idx(0, 0)
    wait_idx(0)
    gather(0)
    fetch_idx(1, 1)

    def group_body(p, carry):
        for b in range(NBUF):
            c = p * NBUF + b
            nxt = (b + 1) % NBUF
            nxt2 = (b + 2) % NBUF
            ob = b % NOB

            @pl.when(c + 1 < NCHUNK)
            def _():
                wait_idx(nxt)
                gather(nxt)

            @pl.when(c + 2 < NCHUNK)
            def _():
                fetch_idx(c + 2, nxt2)

            wait_gather(b)

            @pl.when(c >= NOB)
            def _():
                wait_write(ob)

            p0 = lax.rem(c * CHUNK, MAX_LEN)

            def row_body(r, poff):
                for d in range(NSL):
                    sl = pl.ds(d * LANES, LANES)
                    out_v[ob][r, sl] = (
                        rows_v[b][r, sl] * SCALE
                        + pos_v[pl.ds(poff + d * LANES, LANES)])
                return poff + EMBED

            lax.fori_loop(0, CHUNK, row_body, p0 * EMBED, unroll=8)

            write_out(c, ob)
        return carry

    lax.fori_loop(0, NGRP, group_body, 0, unroll=False)

    for ob in range(NOB):
        wait_write(ob)


def kernel(to_emb, token_table, pos_table):
    idx = to_emb.reshape(ROWS)
    pos_rep = (jnp.concatenate([pos_table] * 3, axis=0)[:MAX_LEN + CHUNK]
               .reshape(-1))
    wide = _widen_kernel(token_table)
    out = _embed_kernel(idx, pos_rep, wide)
    return out.reshape(BATCH, MAX_LEN, EMBED)
